# baseline (speedup 1.0000x reference)
import functools

import jax
import jax.numpy as jnp
from jax import lax
from jax.experimental import pallas as pl
from jax.experimental.pallas import tpu as pltpu
from jax.experimental.pallas import tpu_sc as plsc

_GDN = lax.GatherDimensionNumbers(
    offset_dims=(), collapsed_slice_dims=(0,), start_index_map=(0,))


def _lane_bcast(vec16, lane):
    idx = jnp.full((16, 1), lane, jnp.int32)
    return lax.gather(vec16, idx, _GDN, slice_sizes=(1,),
                      mode=lax.GatherScatterMode.PROMISE_IN_BOUNDS)


N_USERS = 10000
N_ITEMS = 40000
N_NODES = N_USERS + N_ITEMS
D = 64
N_LAYERS = 3
N_EDGES = 800000

NUM_SC = 2
NUM_TILES = 16
K = 128
SUPER = 3
CHUNKS = 393
BODIES = CHUNKS // SUPER
E_PAD = NUM_TILES * CHUNKS * K
ROWS2D = E_PAD // K
HALF = 25088
NP = NUM_SC * HALF
ROWS_PER_TILE = HALF // NUM_TILES

_mesh = plsc.VectorSubcoreMesh(core_axis_name="c", subcore_axis_name="s")


@functools.partial(
    pl.kernel,
    out_type=jax.ShapeDtypeStruct((NP, D), jnp.float32),
    mesh=_mesh,
    compiler_params=pltpu.CompilerParams(needs_layout_passes=False,
                                         use_tc_tiling_on_sc=False),
    scratch_types=[
        pltpu.VMEM((SUPER, K), jnp.int32),
        pltpu.VMEM((SUPER, K), jnp.int32),
        pltpu.VMEM((SUPER, K), jnp.int32),
        pltpu.VMEM((SUPER, K), jnp.float32),
        [pltpu.VMEM((K, D), jnp.float32) for _ in range(SUPER)],
        pltpu.VMEM_SHARED((HALF, D), jnp.float32),
        pltpu.SemaphoreType.DMA((SUPER,)),
        pltpu.SemaphoreType.DMA((SUPER,)),
        pltpu.SemaphoreType.DMA,
    ],
)
def _layer(table, src, dst, w, out,
           src_v, dst_v, loc_v, w_v, rows, acc, gsem, ssem, zsem):
    c = lax.axis_index("c")
    s = lax.axis_index("s")
    zero16 = jnp.zeros((16,), jnp.float32)

    def _zero_rows(j, _):
        for q in range(D // 16):
            rows[0][j, pl.ds(16 * q, 16)] = zero16
        return 0

    lax.fori_loop(0, K, _zero_rows, 0)
    acc_base = pl.multiple_of(s * ROWS_PER_TILE, 8)
    n_full = ROWS_PER_TILE // K
    rem = ROWS_PER_TILE - n_full * K
    zcopies = []
    for j in range(n_full):
        zcopies.append(pltpu.async_copy(
            rows[0], acc.at[pl.ds(acc_base + j * K, K)], zsem))
    zcopies.append(pltpu.async_copy(
        rows[0].at[pl.ds(0, rem)],
        acc.at[pl.ds(acc_base + n_full * K, rem)], zsem))
    for d in zcopies:
        d.wait()
    plsc.subcore_barrier()

    node_base = pl.multiple_of(c * HALF, 8)
    iota16 = lax.iota(jnp.int32, 16)

    def _body(u, _):
        row_base = s * CHUNKS + u * SUPER
        pltpu.sync_copy(src.at[pl.ds(row_base, SUPER)], src_v)
        pltpu.sync_copy(dst.at[pl.ds(row_base, SUPER)], dst_v)
        pltpu.sync_copy(w.at[pl.ds(row_base, SUPER)], w_v)
        for k in range(SUPER):
            @pl.when(u > 0)
            def _drain(k=k):
                pltpu.make_async_copy(
                    rows[k], acc.at[loc_v.at[k]], ssem.at[k]).wait()
        for k in range(SUPER):
            for g in range(K // 16):
                sl = pl.ds(16 * g, 16)
                loc = dst_v[k, sl] - node_base
                in_half = (loc >= 0) & (loc < HALF)
                spread = lax.rem((row_base + k) * K + 16 * g + iota16, HALF)
                loc_v[k, sl] = jnp.where(in_half, loc, spread)
                w_v[k, sl] = jnp.where(in_half, w_v[k, sl], 0.0)
        gathers = []
        for k in range(SUPER):
            gathers.append(pltpu.async_copy(
                table.at[src_v.at[k]], rows[k], gsem.at[k]))
        for k in range(SUPER):
            gathers[k].wait()

            def _scale(g, _, k=k):
                wreg = w_v[k, pl.ds(16 * g, 16)]
                for l in range(16):
                    wb = _lane_bcast(wreg, l)
                    e = 16 * g + l
                    for q in range(D // 16):
                        sl = pl.ds(16 * q, 16)
                        rows[k][e, sl] = rows[k][e, sl] * wb
                return 0

            lax.fori_loop(0, K // 16, _scale, 0)
            pltpu.async_copy(rows[k], acc.at[loc_v.at[k]], ssem.at[k],
                             add=True)
        return 0

    lax.fori_loop(0, BODIES, _body, 0)
    for k in range(SUPER):
        pltpu.make_async_copy(rows[k], acc.at[loc_v.at[k]], ssem.at[k]).wait()
    plsc.subcore_barrier()

    out_base = pl.multiple_of(node_base + acc_base, 8)
    ocopies = []
    for j in range(n_full):
        ocopies.append(pltpu.async_copy(
            acc.at[pl.ds(acc_base + j * K, K)],
            out.at[pl.ds(out_base + j * K, K)], zsem))
    ocopies.append(pltpu.async_copy(
        acc.at[pl.ds(acc_base + n_full * K, rem)],
        out.at[pl.ds(out_base + n_full * K, rem)], zsem))
    for d in ocopies:
        d.wait()


def _mean_kernel(t0, t1, t2, t3, o):
    o[...] = (t0[...] + t1[...] + t2[...] + t3[...]) * 0.25


_N_BLOCKS = 8
_BLOCK = NP // _N_BLOCKS


def _mean4(t0, t1, t2, t3):
    spec = pl.BlockSpec((_BLOCK, D), lambda i: (i, 0))
    return pl.pallas_call(
        _mean_kernel,
        out_shape=jax.ShapeDtypeStruct((NP, D), jnp.float32),
        grid=(_N_BLOCKS,),
        in_specs=[spec] * 4,
        out_specs=spec,
    )(t0, t1, t2, t3)


def kernel(users_emb, items_emb, edge_index, edge_weight):
    table0 = jnp.concatenate(
        [users_emb, items_emb,
         jnp.zeros((NP - N_NODES, D), jnp.float32)], axis=0)
    pad_e = E_PAD - N_EDGES
    src = jnp.concatenate(
        [edge_index[0].astype(jnp.int32),
         jnp.arange(pad_e, dtype=jnp.int32) % N_NODES]).reshape(ROWS2D, K)
    dst = jnp.concatenate(
        [edge_index[1].astype(jnp.int32),
         jnp.zeros((pad_e,), jnp.int32)]).reshape(ROWS2D, K)
    w = jnp.concatenate(
        [edge_weight, jnp.zeros((pad_e,), jnp.float32)]).reshape(ROWS2D, K)

    tables = [table0]
    for _ in range(N_LAYERS):
        tables.append(_layer(tables[-1], src, dst, w))
    light_out = _mean4(*tables)
    return (light_out[:N_USERS], light_out[N_USERS:N_NODES])


# speedup vs baseline: 1.6547x; 1.6547x over previous
import functools

import jax
import jax.numpy as jnp
from jax import lax
from jax.experimental import pallas as pl
from jax.experimental.pallas import tpu as pltpu
from jax.experimental.pallas import tpu_sc as plsc

_GDN = lax.GatherDimensionNumbers(
    offset_dims=(), collapsed_slice_dims=(0,), start_index_map=(0,))


def _lane_bcast(vec16, lane):
    idx = jnp.full((16, 1), lane, jnp.int32)
    return lax.gather(vec16, idx, _GDN, slice_sizes=(1,),
                      mode=lax.GatherScatterMode.PROMISE_IN_BOUNDS)


N_USERS = 10000
N_ITEMS = 40000
N_NODES = N_USERS + N_ITEMS
D = 64
N_LAYERS = 3
N_EDGES = 800000

NUM_SC = 2
NUM_TILES = 16
K = 128
SUPER = 3
CHUNKS = 393
BODIES = CHUNKS // SUPER
E_PAD = NUM_TILES * CHUNKS * K
ROWS2D = E_PAD // K
HALF = 25088
NP = NUM_SC * HALF
ROWS_PER_TILE = HALF // NUM_TILES

_mesh = plsc.VectorSubcoreMesh(core_axis_name="c", subcore_axis_name="s")


@functools.partial(
    pl.kernel,
    out_type=jax.ShapeDtypeStruct((NP, D), jnp.float32),
    mesh=_mesh,
    compiler_params=pltpu.CompilerParams(needs_layout_passes=False,
                                         use_tc_tiling_on_sc=False),
    scratch_types=[
        pltpu.VMEM((SUPER, K), jnp.int32),
        pltpu.VMEM((SUPER, K), jnp.int32),
        pltpu.VMEM((SUPER, K), jnp.int32),
        pltpu.VMEM((SUPER, K), jnp.float32),
        [pltpu.VMEM((K, D), jnp.float32) for _ in range(SUPER)],
        pltpu.VMEM_SHARED((HALF, D), jnp.float32),
        pltpu.SemaphoreType.DMA((SUPER,)),
        pltpu.SemaphoreType.DMA((SUPER,)),
        pltpu.SemaphoreType.DMA,
    ],
)
def _layer(table, src, dst, w, out,
           src_v, dst_v, loc_v, w_v, rows, acc, gsem, ssem, zsem):
    c = lax.axis_index("c")
    s = lax.axis_index("s")
    zero16 = jnp.zeros((16,), jnp.float32)

    def _zero_rows(j, _):
        for q in range(D // 16):
            rows[0][j, pl.ds(16 * q, 16)] = zero16
        return 0

    lax.fori_loop(0, K, _zero_rows, 0)
    acc_base = pl.multiple_of(s * ROWS_PER_TILE, 8)
    n_full = ROWS_PER_TILE // K
    rem = ROWS_PER_TILE - n_full * K
    zcopies = []
    for j in range(n_full):
        zcopies.append(pltpu.async_copy(
            rows[0], acc.at[pl.ds(acc_base + j * K, K)], zsem))
    zcopies.append(pltpu.async_copy(
        rows[0].at[pl.ds(0, rem)],
        acc.at[pl.ds(acc_base + n_full * K, rem)], zsem))
    for d in zcopies:
        d.wait()
    plsc.subcore_barrier()

    node_base = pl.multiple_of(c * HALF, 8)
    iota16 = lax.iota(jnp.int32, 16)

    def _body(u, _):
        row_base = s * CHUNKS + u * SUPER
        pltpu.sync_copy(src.at[pl.ds(row_base, SUPER)], src_v)
        pltpu.sync_copy(dst.at[pl.ds(row_base, SUPER)], dst_v)
        pltpu.sync_copy(w.at[pl.ds(row_base, SUPER)], w_v)
        for k in range(SUPER):
            @pl.when(u > 0)
            def _drain(k=k):
                pltpu.make_async_copy(
                    rows[k], acc.at[loc_v.at[k]], ssem.at[k]).wait()
        for k in range(SUPER):
            for g in range(K // 16):
                sl = pl.ds(16 * g, 16)
                loc = dst_v[k, sl] - node_base
                in_half = (loc >= 0) & (loc < HALF)
                spread = lax.rem((row_base + k) * K + 16 * g + iota16, HALF)
                loc_v[k, sl] = jnp.where(in_half, loc, spread)
                w_v[k, sl] = jnp.where(in_half, w_v[k, sl], 0.0)
        gathers = []
        for k in range(SUPER):
            gathers.append(pltpu.async_copy(
                table.at[src_v.at[k]], rows[k], gsem.at[k]))
        for k in range(SUPER):
            gathers[k].wait()
            for g in range(K // 16):
                wreg = w_v[k, pl.ds(16 * g, 16)]
                for l in range(16):
                    wb = _lane_bcast(wreg, l)
                    e = 16 * g + l
                    for q in range(D // 16):
                        sl = pl.ds(16 * q, 16)
                        rows[k][e, sl] = rows[k][e, sl] * wb
            pltpu.async_copy(rows[k], acc.at[loc_v.at[k]], ssem.at[k],
                             add=True)
        return 0

    lax.fori_loop(0, BODIES, _body, 0)
    for k in range(SUPER):
        pltpu.make_async_copy(rows[k], acc.at[loc_v.at[k]], ssem.at[k]).wait()
    plsc.subcore_barrier()

    out_base = pl.multiple_of(node_base + acc_base, 8)
    ocopies = []
    for j in range(n_full):
        ocopies.append(pltpu.async_copy(
            acc.at[pl.ds(acc_base + j * K, K)],
            out.at[pl.ds(out_base + j * K, K)], zsem))
    ocopies.append(pltpu.async_copy(
        acc.at[pl.ds(acc_base + n_full * K, rem)],
        out.at[pl.ds(out_base + n_full * K, rem)], zsem))
    for d in ocopies:
        d.wait()


def _mean_kernel(t0, t1, t2, t3, o):
    o[...] = (t0[...] + t1[...] + t2[...] + t3[...]) * 0.25


_N_BLOCKS = 8
_BLOCK = NP // _N_BLOCKS


def _mean4(t0, t1, t2, t3):
    spec = pl.BlockSpec((_BLOCK, D), lambda i: (i, 0))
    return pl.pallas_call(
        _mean_kernel,
        out_shape=jax.ShapeDtypeStruct((NP, D), jnp.float32),
        grid=(_N_BLOCKS,),
        in_specs=[spec] * 4,
        out_specs=spec,
    )(t0, t1, t2, t3)


def kernel(users_emb, items_emb, edge_index, edge_weight):
    table0 = jnp.concatenate(
        [users_emb, items_emb,
         jnp.zeros((NP - N_NODES, D), jnp.float32)], axis=0)
    pad_e = E_PAD - N_EDGES
    src = jnp.concatenate(
        [edge_index[0].astype(jnp.int32),
         jnp.arange(pad_e, dtype=jnp.int32) % N_NODES]).reshape(ROWS2D, K)
    dst = jnp.concatenate(
        [edge_index[1].astype(jnp.int32),
         jnp.zeros((pad_e,), jnp.int32)]).reshape(ROWS2D, K)
    w = jnp.concatenate(
        [edge_weight, jnp.zeros((pad_e,), jnp.float32)]).reshape(ROWS2D, K)

    tables = [table0]
    for _ in range(N_LAYERS):
        tables.append(_layer(tables[-1], src, dst, w))
    light_out = _mean4(*tables)
    return (light_out[:N_USERS], light_out[N_USERS:N_NODES])
